# baseline (speedup 1.0000x reference)
import functools

import jax
import jax.numpy as jnp
from jax import lax
from jax.experimental import pallas as pl
from jax.experimental.pallas import tpu as pltpu
from jax.experimental.pallas import tpu_sc as plsc

N = 100000
D = 128
ROWS = 101
CHUNK = 128
L = 16
GROUPS = CHUNK // L

_info = plsc.get_sparse_core_info()
NC, NS = _info.num_cores, _info.num_subcores
NW = NC * NS
TPW = -(-N // (CHUNK * NW))
NBUF = 5
G = TPW // NBUF

_mesh = plsc.VectorSubcoreMesh(core_axis_name="c", subcore_axis_name="s")


@functools.partial(
    pl.kernel,
    mesh=_mesh,
    compiler_params=pltpu.CompilerParams(needs_layout_passes=False),
    out_type=jax.ShapeDtypeStruct((N * D,), jnp.float32),
    scratch_types=(
        [pltpu.VMEM((ROWS * D,), jnp.float32)]
        + [pltpu.VMEM((CHUNK,), jnp.int32) for _ in range(NBUF)]
        + [pltpu.VMEM((CHUNK * D,), jnp.float32) for _ in range(NBUF)]
        + [pltpu.SemaphoreType.DMA((NBUF,)), pltpu.SemaphoreType.DMA((NBUF,))]
    ),
)
def _emb_lookup(z_hbm, table_hbm, out_hbm, table_v, *rest):
    idx_v = rest[:NBUF]
    rows_v = rest[NBUF : 2 * NBUF]
    sem_i, sem_o = rest[2 * NBUF], rest[2 * NBUF + 1]
    wid = lax.axis_index("s") * NC + lax.axis_index("c")

    def base_of(t):
        return jnp.minimum((t * NW + wid) * CHUNK, N - CHUNK)

    def idx_copy(t, b):
        return pltpu.make_async_copy(
            z_hbm.at[pl.ds(base_of(t), CHUNK)], idx_v[b], sem_i.at[b]
        )

    def out_copy(t, b):
        return pltpu.make_async_copy(
            rows_v[b], out_hbm.at[pl.ds(base_of(t) * D, CHUNK * D)], sem_o.at[b]
        )

    pltpu.sync_copy(table_hbm, table_v)

    for b in range(NBUF):
        idx_copy(b, b).start()

    lane = lax.iota(jnp.int32, L)

    def fill_chunk(b):
        def per_group(r, carry):
            z16 = idx_v[b][pl.ds(r * L, L)]
            gbase = z16 * D
            sbase = (r * L + lane) * D
            for c in range(D):
                v = plsc.load_gather(table_v, [gbase + c])
                plsc.store_scatter(rows_v[b], [sbase + c], v)
            return carry

        lax.fori_loop(0, GROUPS, per_group, 0)

    def group(g, carry):
        for b in range(NBUF):
            t = g * NBUF + b
            idx_copy(t, b).wait()

            @pl.when(g > 0)
            def _drain_prev_store():
                out_copy(t, b).wait()

            fill_chunk(b)
            out_copy(t, b).start()

            @pl.when(g < G - 1)
            def _prefetch_idx():
                idx_copy(t + NBUF, b).start()

        return carry

    lax.fori_loop(0, G, group, 0)

    for b in range(NBUF):
        out_copy((G - 1) * NBUF + b, b).wait()


def kernel(z, emb_table):
    return _emb_lookup(z, emb_table.reshape(-1)).reshape(N, D)


# speedup vs baseline: 2.3857x; 2.3857x over previous
import functools

import jax
import jax.numpy as jnp
from jax import lax
from jax.experimental import pallas as pl
from jax.experimental.pallas import tpu as pltpu
from jax.experimental.pallas import tpu_sc as plsc

N = 100000
D = 128
ROWS = 101
CHUNK = 128
L = 16
GROUPS = CHUNK // L

_info = plsc.get_sparse_core_info()
NC, NS = _info.num_cores, _info.num_subcores
NW = NC * NS
TPW = -(-N // (CHUNK * NW))
NBUF = 5
G = TPW // NBUF

_mesh = plsc.VectorSubcoreMesh(core_axis_name="c", subcore_axis_name="s")


@functools.partial(
    pl.kernel,
    mesh=_mesh,
    compiler_params=pltpu.CompilerParams(needs_layout_passes=False),
    out_type=jax.ShapeDtypeStruct((N * D,), jnp.float32),
    scratch_types=(
        [pltpu.VMEM((ROWS * D,), jnp.float32)]
        + [pltpu.VMEM((CHUNK,), jnp.int32) for _ in range(NBUF)]
        + [pltpu.VMEM((CHUNK * D,), jnp.float32) for _ in range(NBUF)]
        + [pltpu.SemaphoreType.DMA((NBUF,)), pltpu.SemaphoreType.DMA((NBUF,))]
    ),
)
def _emb_lookup(z_hbm, table_hbm, out_hbm, table_v, *rest):
    idx_v = rest[:NBUF]
    rows_v = rest[NBUF : 2 * NBUF]
    sem_i, sem_o = rest[2 * NBUF], rest[2 * NBUF + 1]
    wid = lax.axis_index("s") * NC + lax.axis_index("c")

    def base_of(t):
        return jnp.minimum((t * NW + wid) * CHUNK, N - CHUNK)

    def idx_copy(t, b):
        return pltpu.make_async_copy(
            z_hbm.at[pl.ds(base_of(t), CHUNK)], idx_v[b], sem_i.at[b]
        )

    def out_copy(t, b):
        return pltpu.make_async_copy(
            rows_v[b], out_hbm.at[pl.ds(base_of(t) * D, CHUNK * D)], sem_o.at[b]
        )

    pltpu.sync_copy(table_hbm, table_v)

    for b in range(NBUF):
        idx_copy(b, b).start()

    lane = lax.iota(jnp.int32, L)

    def fill_chunk(b):
        def per_group(r, carry):
            z16 = idx_v[b][pl.ds(r * L, L)]
            gbase = z16 * D
            sbase = (r * L + lane) * D

            @plsc.parallel_loop(0, D, unroll=8)
            def _per_col(c):
                v = plsc.load_gather(table_v, [gbase + c])
                plsc.store_scatter(rows_v[b], [sbase + c], v)

            return carry

        lax.fori_loop(0, GROUPS, per_group, 0)

    def group(g, carry):
        for b in range(NBUF):
            t = g * NBUF + b
            idx_copy(t, b).wait()

            @pl.when(g > 0)
            def _drain_prev_store():
                out_copy(t, b).wait()

            fill_chunk(b)
            out_copy(t, b).start()

            @pl.when(g < G - 1)
            def _prefetch_idx():
                idx_copy(t + NBUF, b).start()

        return carry

    lax.fori_loop(0, G, group, 0)

    for b in range(NBUF):
        out_copy((G - 1) * NBUF + b, b).wait()


def kernel(z, emb_table):
    return _emb_lookup(z, emb_table.reshape(-1)).reshape(N, D)


# speedup vs baseline: 4.4561x; 1.8679x over previous
import functools

import jax
import jax.numpy as jnp
from jax import lax
from jax.experimental import pallas as pl
from jax.experimental.pallas import tpu as pltpu
from jax.experimental.pallas import tpu_sc as plsc

N = 100000
D = 128
ROWS = 101
CHUNK = 128
L = 16
VPR = D // L

_info = plsc.get_sparse_core_info()
NC, NS = _info.num_cores, _info.num_subcores
NW = NC * NS
TPW = -(-N // (CHUNK * NW))
NBUF = 5
NGATHER = 2
G = TPW // NBUF

_mesh = plsc.VectorSubcoreMesh(core_axis_name="c", subcore_axis_name="s")


@functools.partial(
    pl.kernel,
    mesh=_mesh,
    compiler_params=pltpu.CompilerParams(needs_layout_passes=False),
    out_type=jax.ShapeDtypeStruct((N, D), jnp.float32),
    scratch_types=(
        [pltpu.VMEM((ROWS, D), jnp.float32)]
        + [pltpu.VMEM((CHUNK,), jnp.int32) for _ in range(NBUF)]
        + [pltpu.VMEM((CHUNK, D), jnp.float32) for _ in range(NBUF)]
        + [
            pltpu.SemaphoreType.DMA((NBUF,)),
            pltpu.SemaphoreType.DMA((NBUF,)),
            pltpu.SemaphoreType.DMA((NBUF,)),
        ]
    ),
)
def _emb_lookup(z_hbm, table_hbm, out_hbm, table_v, *rest):
    idx_v = rest[:NBUF]
    rows_v = rest[NBUF : 2 * NBUF]
    sem_i = rest[2 * NBUF]
    sem_o = rest[2 * NBUF + 1]
    sem_g = rest[2 * NBUF + 2]
    wid = lax.axis_index("s") * NC + lax.axis_index("c")

    def base_of(t):
        return jnp.minimum((t * NW + wid) * CHUNK, N - CHUNK)

    def idx_copy(t, b):
        return pltpu.make_async_copy(
            z_hbm.at[pl.ds(base_of(t), CHUNK)], idx_v[b], sem_i.at[b]
        )

    def out_copy(t, b):
        return pltpu.make_async_copy(
            rows_v[b], out_hbm.at[pl.ds(base_of(t), CHUNK)], sem_o.at[b]
        )

    def gather_copy(b):
        return pltpu.make_async_copy(
            table_hbm.at[idx_v[b]], rows_v[b], sem_g.at[b]
        )

    pltpu.sync_copy(table_hbm, table_v)

    for b in range(NBUF):
        idx_copy(b, b).start()

    def fill_chunk(b):
        @plsc.parallel_loop(0, CHUNK // L, unroll=2)
        def _group(r):
            zvec = idx_v[b][pl.ds(r * L, L)]
            for l in range(L):
                src = zvec[l]
                dst = r * L + l
                for k in range(VPR):
                    rows_v[b][dst, pl.ds(k * L, L)] = table_v[
                        src, pl.ds(k * L, L)
                    ]

    def group(g, carry):
        for b in range(NGATHER):
            t = g * NBUF + b
            idx_copy(t, b).wait()

            @pl.when(g > 0)
            def _drain_prev_store_g():
                out_copy(t, b).wait()

            gather_copy(b).start()

        for b in range(NGATHER, NBUF):
            t = g * NBUF + b
            idx_copy(t, b).wait()

            @pl.when(g > 0)
            def _drain_prev_store_c():
                out_copy(t, b).wait()

            fill_chunk(b)
            out_copy(t, b).start()

            @pl.when(g < G - 1)
            def _prefetch_idx_c():
                idx_copy(t + NBUF, b).start()

        for b in range(NGATHER):
            t = g * NBUF + b
            gather_copy(b).wait()
            out_copy(t, b).start()

            @pl.when(g < G - 1)
            def _prefetch_idx_g():
                idx_copy(t + NBUF, b).start()

        return carry

    lax.fori_loop(0, G, group, 0)

    for b in range(NBUF):
        out_copy((G - 1) * NBUF + b, b).wait()


def kernel(z, emb_table):
    return _emb_lookup(z, emb_table)


# speedup vs baseline: 5.4013x; 1.2121x over previous
import functools

import jax
import jax.numpy as jnp
from jax import lax
from jax.experimental import pallas as pl
from jax.experimental.pallas import tpu as pltpu
from jax.experimental.pallas import tpu_sc as plsc

N = 100000
D = 128
ROWS = 101
CHUNK = 128
L = 16
VPR = D // L

_info = plsc.get_sparse_core_info()
NC, NS = _info.num_cores, _info.num_subcores
NW = NC * NS
TPW = -(-N // (CHUNK * NW))
NBUF = 5
NGATHER = 0
G = TPW // NBUF

_mesh = plsc.VectorSubcoreMesh(core_axis_name="c", subcore_axis_name="s")


@functools.partial(
    pl.kernel,
    mesh=_mesh,
    compiler_params=pltpu.CompilerParams(needs_layout_passes=False),
    out_type=jax.ShapeDtypeStruct((N, D), jnp.float32),
    scratch_types=(
        [pltpu.VMEM((ROWS, D), jnp.float32)]
        + [pltpu.VMEM((CHUNK,), jnp.int32) for _ in range(NBUF)]
        + [pltpu.VMEM((CHUNK, D), jnp.float32) for _ in range(NBUF)]
        + [
            pltpu.SemaphoreType.DMA((NBUF,)),
            pltpu.SemaphoreType.DMA((NBUF,)),
            pltpu.SemaphoreType.DMA((NBUF,)),
        ]
    ),
)
def _emb_lookup(z_hbm, table_hbm, out_hbm, table_v, *rest):
    idx_v = rest[:NBUF]
    rows_v = rest[NBUF : 2 * NBUF]
    sem_i = rest[2 * NBUF]
    sem_o = rest[2 * NBUF + 1]
    sem_g = rest[2 * NBUF + 2]
    wid = lax.axis_index("s") * NC + lax.axis_index("c")

    def base_of(t):
        return jnp.minimum((t * NW + wid) * CHUNK, N - CHUNK)

    def idx_copy(t, b):
        return pltpu.make_async_copy(
            z_hbm.at[pl.ds(base_of(t), CHUNK)], idx_v[b], sem_i.at[b]
        )

    def out_copy(t, b):
        return pltpu.make_async_copy(
            rows_v[b], out_hbm.at[pl.ds(base_of(t), CHUNK)], sem_o.at[b]
        )

    def gather_copy(b):
        return pltpu.make_async_copy(
            table_hbm.at[idx_v[b]], rows_v[b], sem_g.at[b]
        )

    pltpu.sync_copy(table_hbm, table_v)

    for b in range(NBUF):
        idx_copy(b, b).start()

    def fill_chunk(b):
        @plsc.parallel_loop(0, CHUNK // L, unroll=2)
        def _group(r):
            zvec = idx_v[b][pl.ds(r * L, L)]
            for l in range(L):
                src = zvec[l]
                dst = r * L + l
                for k in range(VPR):
                    rows_v[b][dst, pl.ds(k * L, L)] = table_v[
                        src, pl.ds(k * L, L)
                    ]

    def group(g, carry):
        for b in range(NGATHER):
            t = g * NBUF + b
            idx_copy(t, b).wait()

            @pl.when(g > 0)
            def _drain_prev_store_g():
                out_copy(t, b).wait()

            gather_copy(b).start()

        for b in range(NGATHER, NBUF):
            t = g * NBUF + b
            idx_copy(t, b).wait()

            @pl.when(g > 0)
            def _drain_prev_store_c():
                out_copy(t, b).wait()

            fill_chunk(b)
            out_copy(t, b).start()

            @pl.when(g < G - 1)
            def _prefetch_idx_c():
                idx_copy(t + NBUF, b).start()

        for b in range(NGATHER):
            t = g * NBUF + b
            gather_copy(b).wait()
            out_copy(t, b).start()

            @pl.when(g < G - 1)
            def _prefetch_idx_g():
                idx_copy(t + NBUF, b).start()

        return carry

    lax.fori_loop(0, G, group, 0)

    for b in range(NBUF):
        out_copy((G - 1) * NBUF + b, b).wait()


def kernel(z, emb_table):
    return _emb_lookup(z, emb_table)


# speedup vs baseline: 10.6214x; 1.9665x over previous
import functools

import jax
import jax.numpy as jnp
from jax import lax
from jax.experimental import pallas as pl
from jax.experimental.pallas import tpu as pltpu
from jax.experimental.pallas import tpu_sc as plsc

N = 100000
D = 128
ROWS = 101
CHUNK = 128
L = 16
VPR = D // L

_info = plsc.get_sparse_core_info()
NC, NS = _info.num_cores, _info.num_subcores
NW = NC * NS
TPW = -(-N // (CHUNK * NW))
NBUF = 5
NGATHER = 5
G = TPW // NBUF

_mesh = plsc.VectorSubcoreMesh(core_axis_name="c", subcore_axis_name="s")


@functools.partial(
    pl.kernel,
    mesh=_mesh,
    compiler_params=pltpu.CompilerParams(needs_layout_passes=False),
    out_type=jax.ShapeDtypeStruct((N, D), jnp.float32),
    scratch_types=(
        [pltpu.VMEM((ROWS, D), jnp.float32)]
        + [pltpu.VMEM_SHARED((ROWS, D), jnp.float32)]
        + [pltpu.VMEM((CHUNK,), jnp.int32) for _ in range(NBUF)]
        + [pltpu.VMEM((CHUNK, D), jnp.float32) for _ in range(NBUF)]
        + [
            pltpu.SemaphoreType.DMA((NBUF,)),
            pltpu.SemaphoreType.DMA((NBUF,)),
            pltpu.SemaphoreType.DMA((NBUF,)),
        ]
    ),
)
def _emb_lookup(z_hbm, table_hbm, out_hbm, table_v, table_sh, *rest):
    idx_v = rest[:NBUF]
    rows_v = rest[NBUF : 2 * NBUF]
    sem_i = rest[2 * NBUF]
    sem_o = rest[2 * NBUF + 1]
    sem_g = rest[2 * NBUF + 2]
    wid = lax.axis_index("s") * NC + lax.axis_index("c")

    def base_of(t):
        return jnp.minimum((t * NW + wid) * CHUNK, N - CHUNK)

    def idx_copy(t, b):
        return pltpu.make_async_copy(
            z_hbm.at[pl.ds(base_of(t), CHUNK)], idx_v[b], sem_i.at[b]
        )

    def out_copy(t, b):
        return pltpu.make_async_copy(
            rows_v[b], out_hbm.at[pl.ds(base_of(t), CHUNK)], sem_o.at[b]
        )

    def gather_copy(b):
        return pltpu.make_async_copy(
            table_sh.at[idx_v[b]], rows_v[b], sem_g.at[b]
        )

    pltpu.sync_copy(table_hbm, table_v)

    @pl.when(lax.axis_index("s") == 0)
    def _stage_shared():
        pltpu.sync_copy(table_hbm, table_sh)

    plsc.subcore_barrier()

    for b in range(NBUF):
        idx_copy(b, b).start()

    def fill_chunk(b):
        @plsc.parallel_loop(0, CHUNK // L, unroll=2)
        def _group(r):
            zvec = idx_v[b][pl.ds(r * L, L)]
            for l in range(L):
                src = zvec[l]
                dst = r * L + l
                for k in range(VPR):
                    rows_v[b][dst, pl.ds(k * L, L)] = table_v[
                        src, pl.ds(k * L, L)
                    ]

    def group(g, carry):
        for b in range(NGATHER):
            t = g * NBUF + b
            idx_copy(t, b).wait()

            @pl.when(g > 0)
            def _drain_prev_store_g():
                out_copy(t, b).wait()

            gather_copy(b).start()

        for b in range(NGATHER, NBUF):
            t = g * NBUF + b
            idx_copy(t, b).wait()

            @pl.when(g > 0)
            def _drain_prev_store_c():
                out_copy(t, b).wait()

            fill_chunk(b)
            out_copy(t, b).start()

            @pl.when(g < G - 1)
            def _prefetch_idx_c():
                idx_copy(t + NBUF, b).start()

        for b in range(NGATHER):
            t = g * NBUF + b
            gather_copy(b).wait()
            out_copy(t, b).start()

            @pl.when(g < G - 1)
            def _prefetch_idx_g():
                idx_copy(t + NBUF, b).start()

        return carry

    lax.fori_loop(0, G, group, 0)

    for b in range(NBUF):
        out_copy((G - 1) * NBUF + b, b).wait()


def kernel(z, emb_table):
    return _emb_lookup(z, emb_table)
